# Initial kernel scaffold; baseline (speedup 1.0000x reference)
import functools

import jax
import jax.numpy as jnp
import numpy as np
from jax import lax
from jax.experimental import pallas as pl
from jax.experimental.pallas import tpu as pltpu
from jax.experimental.pallas import tpu_sc as plsc

_N = 1_000_000
_D = 16
_NZ = 500_000
_NC = 2
_NS = 16
_NW = _NC * _NS
_CH = 128


def _build_index_table() -> np.ndarray:
    idx = np.sort(np.asarray(jax.random.permutation(jax.random.key(42), _N)[:_NZ]))
    per_w = -(-_NZ // _NW)
    n_chunks = -(-per_w // _CH)
    k = n_chunks * _CH
    tab = np.empty((_NW, k), np.int32)
    for w in range(_NW):
        part = idx[w * per_w:(w + 1) * per_w]
        tab[w, : len(part)] = part
        tab[w, len(part):] = part[-1]
    return tab.reshape(_NW, n_chunks, _CH)


_IDX_TAB = _build_index_table()
_N_CHUNKS = _IDX_TAB.shape[1]

_mesh = plsc.VectorSubcoreMesh(core_axis_name="c", subcore_axis_name="s")


@functools.partial(
    pl.kernel,
    mesh=_mesh,
    scratch_types=[
        pltpu.VMEM((_N_CHUNKS, _CH), jnp.int32),
        pltpu.VMEM((_CH, _D), jnp.float32),
        pltpu.SemaphoreType.DMA,
        pltpu.SemaphoreType.DMA,
    ],
)
def _sc_zero_rows(idx_hbm, zeros_hbm, out1, out2, idx_v, zeros_v, lsem, ssem):
    c = lax.axis_index("c")
    s = lax.axis_index("s")
    wid = s * _NC + c

    pltpu.async_copy(zeros_hbm, zeros_v, lsem).wait()
    pltpu.async_copy(idx_hbm.at[wid], idx_v, lsem).wait()

    @pl.loop(0, _N_CHUNKS)
    def _fire(j):
        pltpu.async_copy(zeros_v, out1.at[idx_v.at[j]], ssem)
        pltpu.async_copy(zeros_v, out2.at[idx_v.at[j]], ssem)

    @pl.loop(0, _N_CHUNKS)
    def _drain(j):
        pltpu.make_async_copy(zeros_v, out1.at[idx_v.at[j]], ssem).wait()
        pltpu.make_async_copy(zeros_v, out2.at[idx_v.at[j]], ssem).wait()


def kernel(emb1, emb2):
    idx_tab = jnp.asarray(_IDX_TAB)
    zeros = jnp.zeros((_CH, _D), jnp.float32)
    out1 = jax.new_ref(emb1)
    out2 = jax.new_ref(emb2)
    _sc_zero_rows(idx_tab, zeros, out1, out2)
    return out1[...], out2[...]


# speedup vs baseline: 4.5953x; 4.5953x over previous
import functools

import jax
import jax.numpy as jnp
import numpy as np
from jax import lax
from jax.experimental import pallas as pl
from jax.experimental.pallas import tpu as pltpu
from jax.experimental.pallas import tpu_sc as plsc

_N = 1_000_000
_D = 16
_NZ = 500_000
_NC = 2
_NS = 16
_NW = _NC * _NS
_CH = 128


_PER_W = -(-_NZ // _NW)
_N_CHUNKS = -(-_PER_W // _CH)


@functools.cache
def _build_index_table() -> np.ndarray:
    idx = np.sort(np.asarray(jax.random.permutation(jax.random.key(42), _N)[:_NZ]))
    k = _N_CHUNKS * _CH
    tab = np.empty((_NW, k), np.int32)
    for w in range(_NW):
        part = idx[w * _PER_W:(w + 1) * _PER_W]
        tab[w, : len(part)] = part
        tab[w, len(part):] = part[-1]
    return tab.reshape(_NW, _N_CHUNKS, _CH)


try:
    _build_index_table()
except Exception:
    pass

@functools.cache
def _get_sc_zero_rows():
    mesh = plsc.VectorSubcoreMesh(
        core_axis_name="c", subcore_axis_name="s", num_cores=_NC, num_subcores=_NS
    )

    @functools.partial(
        pl.kernel,
        mesh=mesh,
        compiler_params=pltpu.CompilerParams(use_tc_tiling_on_sc=False),
        scratch_types=[
            pltpu.VMEM((_N_CHUNKS, _CH), jnp.int32),
            pltpu.VMEM((_CH, _D), jnp.float32),
            pltpu.SemaphoreType.DMA,
            pltpu.SemaphoreType.DMA,
        ],
    )
    def _sc_zero_rows(idx_hbm, zeros_hbm, out1, out2, idx_v, zeros_v, lsem, ssem):
        c = lax.axis_index("c")
        s = lax.axis_index("s")
        wid = s * _NC + c

        pltpu.async_copy(zeros_hbm, zeros_v, lsem).wait()
        pltpu.async_copy(idx_hbm.at[wid], idx_v, lsem).wait()

        @pl.loop(0, _N_CHUNKS)
        def _fire(j):
            pltpu.async_copy(zeros_v, out1.at[idx_v.at[j]], ssem)
            pltpu.async_copy(zeros_v, out2.at[idx_v.at[j]], ssem)

        @pl.loop(0, _N_CHUNKS)
        def _drain(j):
            pltpu.make_async_copy(zeros_v, out1.at[idx_v.at[j]], ssem).wait()
            pltpu.make_async_copy(zeros_v, out2.at[idx_v.at[j]], ssem).wait()

    return _sc_zero_rows


def kernel(emb1, emb2):
    idx_tab = jnp.asarray(_build_index_table())
    zeros = jnp.zeros((_CH, _D), jnp.float32)
    out1 = jax.new_ref(emb1)
    out2 = jax.new_ref(emb2)
    _get_sc_zero_rows()(idx_tab, zeros, out1, out2)
    return out1[...], out2[...]
